# baseline (speedup 1.0000x reference)
import functools

import jax
import jax.numpy as jnp
from jax import lax
from jax.experimental import pallas as pl
from jax.experimental.pallas import tpu as pltpu
from jax.experimental.pallas import tpu_sc as plsc

NHID = 128
HALF = 64
NPER = 16384
E = 262144
NC = 2
NS = 16
CH = 128
EDGES_PER_TILE = E // NS
NCHUNK = EDGES_PER_TILE // CH
ROWS_PER_TILE = NPER // NS


NBUF = 4


def _sc_segsum_body(table, src_hbm, dst_hbm, out, sidx, didx, rows, acc,
                    gsem, ssem):
    c = lax.axis_index("c")
    s = lax.axis_index("s")

    zero = jnp.zeros((16,), jnp.float32)

    def zrow(i, carry):
        for j in range(HALF // 16):
            rows[0][i, pl.ds(j * 16, 16)] = zero
        return carry

    lax.fori_loop(0, CH, zrow, 0)
    for k in range(ROWS_PER_TILE // CH):
        pltpu.sync_copy(rows[0], acc.at[pl.ds(s * ROWS_PER_TILE + k * CH, CH)])

    pltpu.sync_copy(src_hbm.at[pl.ds(s * EDGES_PER_TILE, EDGES_PER_TILE)],
                    sidx)
    pltpu.sync_copy(dst_hbm.at[pl.ds(s * NCHUNK, NCHUNK)], didx)
    plsc.subcore_barrier()

    tabc = table.at[c]
    for b in range(NBUF - 1):
        pltpu.async_copy(tabc.at[sidx.at[pl.ds(b * CH, CH)]], rows[b],
                         gsem[b])

    def step(k2, carry):
        for b in range(NBUF):
            k = k2 * NBUF + b
            pltpu.make_async_copy(tabc.at[sidx.at[pl.ds(0, CH)]], rows[b],
                                  gsem[b]).wait()
            pltpu.async_copy(rows[b], acc.at[didx.at[k]], ssem[b], add=True)
            b3 = (b - 1) % NBUF
            kn = k + NBUF - 1

            @pl.when(kn < NCHUNK)
            def _():
                @pl.when(k >= 1)
                def _():
                    pltpu.make_async_copy(rows[b3], acc.at[didx.at[0]],
                                          ssem[b3]).wait()
                pltpu.async_copy(tabc.at[sidx.at[pl.ds(kn * CH, CH)]],
                                 rows[b3], gsem[b3])

        return carry

    lax.fori_loop(0, NCHUNK // NBUF, step, 0)
    for b in range(NBUF):
        pltpu.make_async_copy(rows[b], acc.at[didx.at[0]], ssem[b]).wait()
    plsc.subcore_barrier()
    pltpu.sync_copy(acc.at[pl.ds(s * ROWS_PER_TILE, ROWS_PER_TILE)],
                    out.at[pl.ds(c * NPER + s * ROWS_PER_TILE,
                                 ROWS_PER_TILE)])


@functools.cache
def _get_sc_segsum():
    return functools.partial(
        pl.kernel,
        mesh=plsc.VectorSubcoreMesh(core_axis_name="c", subcore_axis_name="s"),
        out_type=jax.ShapeDtypeStruct((NC * NPER, HALF), jnp.float32),
        scratch_types=[
            pltpu.VMEM((EDGES_PER_TILE,), jnp.int32),
            pltpu.VMEM((NCHUNK, CH), jnp.int32),
            [pltpu.VMEM((CH, HALF), jnp.float32)] * NBUF,
            pltpu.VMEM_SHARED((NPER, HALF), jnp.float32),
            [pltpu.SemaphoreType.DMA] * NBUF,
            [pltpu.SemaphoreType.DMA] * NBUF,
        ],
        compiler_params=pltpu.CompilerParams(use_tc_tiling_on_sc=False),
    )(_sc_segsum_body)


def _sc_segsum(table, src_idx2, dst_idx2):
    return _get_sc_segsum()(table, src_idx2, dst_idx2)


def _mlp_body(a_ref, xres_ref, w0_ref, g0_ref, b0_ref, w1_ref, g1_ref,
              b1_ref, eps_ref, split_ref, full_ref):
    a0 = a_ref[0:NPER, :]
    a1 = a_ref[NPER:2 * NPER, :]
    w0 = w0_ref[...]
    h = (lax.dot_general(a0, w0[:, 0:HALF], (((1,), (1,)), ((), ())),
                         preferred_element_type=jnp.float32)
         + lax.dot_general(a1, w0[:, HALF:NHID], (((1,), (1,)), ((), ())),
                           preferred_element_type=jnp.float32))
    m = jnp.mean(h, axis=0, keepdims=True)
    hm = h - m
    v = jnp.mean(hm * hm, axis=0, keepdims=True)
    x1 = jnp.maximum(hm * lax.rsqrt(v + 1e-5) * g0_ref[...] + b0_ref[...], 0.0)
    h2 = lax.dot_general(x1, w1_ref[...], (((1,), (1,)), ((), ())),
                         preferred_element_type=jnp.float32)
    m2 = jnp.mean(h2, axis=0, keepdims=True)
    hm2 = h2 - m2
    v2 = jnp.mean(hm2 * hm2, axis=0, keepdims=True)
    y = jnp.maximum(hm2 * lax.rsqrt(v2 + 1e-5) * g1_ref[...] + b1_ref[...], 0.0)
    out = (1.0 + eps_ref[0, 0]) * xres_ref[...] + y
    full_ref[...] = out
    split_ref[0:NPER, :] = out[:, 0:HALF]
    split_ref[NPER:2 * NPER, :] = out[:, HALF:NHID]


_mlp = pl.pallas_call(
    _mlp_body,
    out_shape=(jax.ShapeDtypeStruct((2 * NPER, HALF), jnp.float32),
               jax.ShapeDtypeStruct((NPER, NHID), jnp.float32)),
    in_specs=[pl.BlockSpec(memory_space=pltpu.VMEM)] * 8
             + [pl.BlockSpec(memory_space=pltpu.SMEM)],
)


def _split(x):
    n = x.shape[0]
    return x.reshape(n, 2, HALF).transpose(1, 0, 2).reshape(2 * n, HALF)


def _phase(tab_split, x_res_full, src_idx, dst_idx, W0, g0, b0, W1, g1, b1,
           eps):
    tab3 = tab_split.reshape(NC, NPER, HALF)
    dst2 = dst_idx.reshape(NS * NCHUNK, CH)
    agg = _sc_segsum(tab3, src_idx, dst2)
    return _mlp(agg, x_res_full,
                W0, g0.reshape(1, NHID), b0.reshape(1, NHID),
                W1, g1.reshape(1, NHID), b1.reshape(1, NHID),
                eps.reshape(1, 1))


def kernel(xs, k_batch, bipartites, bw_W0, bw_g0, bw_b0, bw_W1, bw_g1, bw_b1,
           fw_W0, fw_g0, fw_b0, fw_W1, fw_g1, fw_b1, bw_eps, fw_eps):
    L0 = xs[0:NPER]
    L1 = xs[NPER:2 * NPER]
    L2 = xs[2 * NPER:3 * NPER]
    ei0 = bipartites[0]
    ei1 = bipartites[1]

    nl1_s, nl1_f = _phase(_split(L2), L1, ei1[1], ei1[0],
                          bw_W0[1], bw_g0[1], bw_b0[1], bw_W1[1], bw_g1[1],
                          bw_b1[1], bw_eps[1])
    nl0_s, nl0_f = _phase(nl1_s, L0, ei0[1], ei0[0],
                          bw_W0[0], bw_g0[0], bw_b0[0], bw_W1[0], bw_g1[0],
                          bw_b1[0], bw_eps[0])
    nl1b_s, nl1b_f = _phase(nl0_s, nl1_f, ei0[0], ei0[1],
                            fw_W0[0], fw_g0[0], fw_b0[0], fw_W1[0], fw_g1[0],
                            fw_b1[0], fw_eps[0])
    _, nl2_f = _phase(nl1b_s, L2, ei1[0], ei1[1],
                      fw_W0[1], fw_g0[1], fw_b0[1], fw_W1[1], fw_g1[1],
                      fw_b1[1], fw_eps[1])
    return jnp.concatenate([nl0_f, nl1b_f, nl2_f], axis=0)


# speedup vs baseline: 1.2146x; 1.2146x over previous
import functools

import jax
import jax.numpy as jnp
from jax import lax
from jax.experimental import pallas as pl
from jax.experimental.pallas import tpu as pltpu
from jax.experimental.pallas import tpu_sc as plsc

NHID = 128
HALF = 64
NPER = 16384
E = 262144
NC = 2
NS = 16
CH = 128
EDGES_PER_TILE = E // NS
NCHUNK = EDGES_PER_TILE // CH
ROWS_PER_TILE = NPER // NS


NBUF = 4


def _sc_segsum_body(table, src_hbm, dst_hbm, out, sidx, didx, rows, acc,
                    gsem, ssem):
    c = lax.axis_index("c")
    s = lax.axis_index("s")

    zero = jnp.zeros((16,), jnp.float32)

    def zrow(i, carry):
        for j in range(HALF // 16):
            rows[0][i, pl.ds(j * 16, 16)] = zero
        return carry

    lax.fori_loop(0, CH, zrow, 0)
    for k in range(ROWS_PER_TILE // CH):
        pltpu.sync_copy(rows[0], acc.at[pl.ds(s * ROWS_PER_TILE + k * CH, CH)])

    pltpu.sync_copy(src_hbm.at[pl.ds(s * EDGES_PER_TILE, EDGES_PER_TILE)],
                    sidx)
    pltpu.sync_copy(dst_hbm.at[pl.ds(s * NCHUNK, NCHUNK)], didx)
    plsc.subcore_barrier()

    tabc = table.at[c]
    for b in range(NBUF - 1):
        pltpu.async_copy(tabc.at[sidx.at[pl.ds(b * CH, CH)]], rows[b],
                         gsem[b])

    def step(k2, carry):
        for b in range(NBUF):
            k = k2 * NBUF + b
            pltpu.make_async_copy(tabc.at[sidx.at[pl.ds(0, CH)]], rows[b],
                                  gsem[b]).wait()
            pltpu.async_copy(rows[b], acc.at[didx.at[k]], ssem[b], add=True)
            b3 = (b - 1) % NBUF
            kn = k + NBUF - 1

            @pl.when(kn < NCHUNK)
            def _():
                @pl.when(k >= 1)
                def _():
                    pltpu.make_async_copy(rows[b3], acc.at[didx.at[0]],
                                          ssem[b3]).wait()
                pltpu.async_copy(tabc.at[sidx.at[pl.ds(kn * CH, CH)]],
                                 rows[b3], gsem[b3])

        return carry

    lax.fori_loop(0, NCHUNK // NBUF, step, 0)
    for b in range(NBUF):
        pltpu.make_async_copy(rows[b], acc.at[didx.at[0]], ssem[b]).wait()
    plsc.subcore_barrier()
    pltpu.sync_copy(acc.at[pl.ds(s * ROWS_PER_TILE, ROWS_PER_TILE)],
                    out.at[pl.ds(c * NPER + s * ROWS_PER_TILE,
                                 ROWS_PER_TILE)])


@functools.cache
def _get_sc_segsum():
    return functools.partial(
        pl.kernel,
        mesh=plsc.VectorSubcoreMesh(core_axis_name="c", subcore_axis_name="s"),
        out_type=jax.ShapeDtypeStruct((NC * NPER, HALF), jnp.float32),
        scratch_types=[
            pltpu.VMEM((EDGES_PER_TILE,), jnp.int32),
            pltpu.VMEM((NCHUNK, CH), jnp.int32),
            [pltpu.VMEM((CH, HALF), jnp.float32)] * NBUF,
            pltpu.VMEM_SHARED((NPER, HALF), jnp.float32),
            [pltpu.SemaphoreType.DMA] * NBUF,
            [pltpu.SemaphoreType.DMA] * NBUF,
        ],
        compiler_params=pltpu.CompilerParams(use_tc_tiling_on_sc=False),
    )(_sc_segsum_body)


def _sc_segsum(table, src_idx2, dst_idx2):
    return _get_sc_segsum()(table, src_idx2, dst_idx2)


SHLF = NPER // 2


def _dot_t(x, w):
    return lax.dot_general(x, w, (((1,), (1,)), ((), ())),
                           preferred_element_type=jnp.float32)


def _mlp_body(a_ref, xres_ref, w0_ref, g0_ref, b0_ref, w1_ref, g1_ref,
              b1_ref, eps_ref, split_ref, full_ref):
    e0 = a_ref[0:SHLF, 0:HALF]
    o0 = a_ref[0:SHLF, HALF:NHID]
    e1 = a_ref[SHLF:NPER, 0:HALF]
    o1 = a_ref[SHLF:NPER, HALF:NHID]
    w0 = w0_ref[...]
    w0t = w0[:, 0:HALF]
    w0b = w0[:, HALF:NHID]
    h_e = _dot_t(e0, w0t) + _dot_t(e1, w0b)
    h_o = _dot_t(o0, w0t) + _dot_t(o1, w0b)
    n = float(NPER)
    m = (jnp.sum(h_e, 0, keepdims=True) + jnp.sum(h_o, 0, keepdims=True)) / n
    hm_e = h_e - m
    hm_o = h_o - m
    v = (jnp.sum(hm_e * hm_e, 0, keepdims=True)
         + jnp.sum(hm_o * hm_o, 0, keepdims=True)) / n
    r0 = lax.rsqrt(v + 1e-5) * g0_ref[...]
    x1_e = jnp.maximum(hm_e * r0 + b0_ref[...], 0.0)
    x1_o = jnp.maximum(hm_o * r0 + b0_ref[...], 0.0)
    w1 = w1_ref[...]
    h2_e = _dot_t(x1_e, w1)
    h2_o = _dot_t(x1_o, w1)
    m2 = (jnp.sum(h2_e, 0, keepdims=True)
          + jnp.sum(h2_o, 0, keepdims=True)) / n
    hm2_e = h2_e - m2
    hm2_o = h2_o - m2
    v2 = (jnp.sum(hm2_e * hm2_e, 0, keepdims=True)
          + jnp.sum(hm2_o * hm2_o, 0, keepdims=True)) / n
    r1 = lax.rsqrt(v2 + 1e-5) * g1_ref[...]
    y_e = jnp.maximum(hm2_e * r1 + b1_ref[...], 0.0)
    y_o = jnp.maximum(hm2_o * r1 + b1_ref[...], 0.0)
    sc = 1.0 + eps_ref[0, 0]
    xr = xres_ref[...]
    out_e = sc * xr[:, 0:NHID] + y_e
    out_o = sc * xr[:, NHID:2 * NHID] + y_o
    full_ref[...] = jnp.concatenate([out_e, out_o], axis=1)
    split_ref[0:SHLF, :] = jnp.concatenate(
        [out_e[:, 0:HALF], out_o[:, 0:HALF]], axis=1)
    split_ref[SHLF:NPER, :] = jnp.concatenate(
        [out_e[:, HALF:NHID], out_o[:, HALF:NHID]], axis=1)


_mlp = pl.pallas_call(
    _mlp_body,
    out_shape=(jax.ShapeDtypeStruct((NPER, NHID), jnp.float32),
               jax.ShapeDtypeStruct((SHLF, 2 * NHID), jnp.float32)),
    in_specs=[pl.BlockSpec(memory_space=pltpu.VMEM)] * 8
             + [pl.BlockSpec(memory_space=pltpu.SMEM)],
)


def _split(x):
    n = x.shape[0]
    return x.reshape(n, 2, HALF).transpose(1, 0, 2).reshape(2 * n, HALF)


def _phase(tab_b, x_res_full, src_idx, dst_idx, W0, g0, b0, W1, g1, b1,
           eps):
    tab3 = tab_b.reshape(NC, NPER, HALF)
    dst2 = dst_idx.reshape(NS * NCHUNK, CH)
    agg = _sc_segsum(tab3, src_idx, dst2)
    split_b, full_v = _mlp(agg.reshape(NPER, NHID),
                           x_res_full.reshape(SHLF, 2 * NHID),
                           W0, g0.reshape(1, NHID), b0.reshape(1, NHID),
                           W1, g1.reshape(1, NHID), b1.reshape(1, NHID),
                           eps.reshape(1, 1))
    return split_b, full_v.reshape(NPER, NHID)


def kernel(xs, k_batch, bipartites, bw_W0, bw_g0, bw_b0, bw_W1, bw_g1, bw_b1,
           fw_W0, fw_g0, fw_b0, fw_W1, fw_g1, fw_b1, bw_eps, fw_eps):
    L0 = xs[0:NPER]
    L1 = xs[NPER:2 * NPER]
    L2 = xs[2 * NPER:3 * NPER]
    ei0 = bipartites[0]
    ei1 = bipartites[1]

    nl1_s, nl1_f = _phase(_split(L2), L1, ei1[1], ei1[0],
                          bw_W0[1], bw_g0[1], bw_b0[1], bw_W1[1], bw_g1[1],
                          bw_b1[1], bw_eps[1])
    nl0_s, nl0_f = _phase(nl1_s, L0, ei0[1], ei0[0],
                          bw_W0[0], bw_g0[0], bw_b0[0], bw_W1[0], bw_g1[0],
                          bw_b1[0], bw_eps[0])
    nl1b_s, nl1b_f = _phase(nl0_s, nl1_f, ei0[0], ei0[1],
                            fw_W0[0], fw_g0[0], fw_b0[0], fw_W1[0], fw_g1[0],
                            fw_b1[0], fw_eps[0])
    _, nl2_f = _phase(nl1b_s, L2, ei1[0], ei1[1],
                      fw_W0[1], fw_g0[1], fw_b0[1], fw_W1[1], fw_g1[1],
                      fw_b1[1], fw_eps[1])
    return jnp.concatenate([nl0_f, nl1b_f, nl2_f], axis=0)


# speedup vs baseline: 1.2360x; 1.0176x over previous
import functools

import jax
import jax.numpy as jnp
from jax import lax
from jax.experimental import pallas as pl
from jax.experimental.pallas import tpu as pltpu
from jax.experimental.pallas import tpu_sc as plsc

NHID = 128
HALF = 64
NPER = 16384
E = 262144
NC = 2
NS = 16
CH = 128
EDGES_PER_TILE = E // NS
NCHUNK = EDGES_PER_TILE // CH
ROWS_PER_TILE = NPER // NS


NBUF = 4


def _sc_segsum_body(table, src_hbm, dst_hbm, out, sidx, didx, rows, acc,
                    gsem, ssem):
    c = lax.axis_index("c")
    s = lax.axis_index("s")

    pltpu.sync_copy(src_hbm.at[pl.ds(s * EDGES_PER_TILE, EDGES_PER_TILE)],
                    sidx)
    pltpu.sync_copy(dst_hbm.at[pl.ds(s * NCHUNK, NCHUNK)], didx)
    tabc = table.at[c]
    for b in range(NBUF - 1):
        pltpu.async_copy(tabc.at[sidx.at[pl.ds(b * CH, CH)]], rows[b],
                         gsem[b])

    zero = jnp.zeros((16,), jnp.float32)
    zbuf = rows[NBUF - 1]

    def zrow(i, carry):
        for j in range(HALF // 16):
            zbuf[i, pl.ds(j * 16, 16)] = zero
        return carry

    lax.fori_loop(0, CH, zrow, 0)
    for k in range(ROWS_PER_TILE // CH):
        pltpu.sync_copy(zbuf, acc.at[pl.ds(s * ROWS_PER_TILE + k * CH, CH)])
    plsc.subcore_barrier()

    def step(k2, carry):
        for b in range(NBUF):
            k = k2 * NBUF + b
            pltpu.make_async_copy(tabc.at[sidx.at[pl.ds(0, CH)]], rows[b],
                                  gsem[b]).wait()
            pltpu.async_copy(rows[b], acc.at[didx.at[k]], ssem[b], add=True)
            b3 = (b - 1) % NBUF
            kn = k + NBUF - 1

            @pl.when(kn < NCHUNK)
            def _():
                @pl.when(k >= 1)
                def _():
                    pltpu.make_async_copy(rows[b3], acc.at[didx.at[0]],
                                          ssem[b3]).wait()
                pltpu.async_copy(tabc.at[sidx.at[pl.ds(kn * CH, CH)]],
                                 rows[b3], gsem[b3])

        return carry

    lax.fori_loop(0, NCHUNK // NBUF, step, 0)
    for b in range(NBUF):
        pltpu.make_async_copy(rows[b], acc.at[didx.at[0]], ssem[b]).wait()
    plsc.subcore_barrier()
    pltpu.sync_copy(acc.at[pl.ds(s * ROWS_PER_TILE, ROWS_PER_TILE)],
                    out.at[pl.ds(c * NPER + s * ROWS_PER_TILE,
                                 ROWS_PER_TILE)])


@functools.cache
def _get_sc_segsum():
    return functools.partial(
        pl.kernel,
        mesh=plsc.VectorSubcoreMesh(core_axis_name="c", subcore_axis_name="s"),
        out_type=jax.ShapeDtypeStruct((NC * NPER, HALF), jnp.float32),
        scratch_types=[
            pltpu.VMEM((EDGES_PER_TILE,), jnp.int32),
            pltpu.VMEM((NCHUNK, CH), jnp.int32),
            [pltpu.VMEM((CH, HALF), jnp.float32)] * NBUF,
            pltpu.VMEM_SHARED((NPER, HALF), jnp.float32),
            [pltpu.SemaphoreType.DMA] * NBUF,
            [pltpu.SemaphoreType.DMA] * NBUF,
        ],
        compiler_params=pltpu.CompilerParams(use_tc_tiling_on_sc=False),
    )(_sc_segsum_body)


def _sc_segsum(table, src_idx2, dst_idx2):
    return _get_sc_segsum()(table, src_idx2, dst_idx2)


SHLF = NPER // 2


def _dot_t(x, w):
    return lax.dot_general(x, w, (((1,), (1,)), ((), ())),
                           preferred_element_type=jnp.float32)


def _mlp_body(a_ref, xres_ref, w0_ref, g0_ref, b0_ref, w1_ref, g1_ref,
              b1_ref, eps_ref, split_ref, full_ref):
    e0 = a_ref[0:SHLF, 0:HALF]
    o0 = a_ref[0:SHLF, HALF:NHID]
    e1 = a_ref[SHLF:NPER, 0:HALF]
    o1 = a_ref[SHLF:NPER, HALF:NHID]
    w0 = w0_ref[...]
    w0t = w0[:, 0:HALF]
    w0b = w0[:, HALF:NHID]
    h_e = _dot_t(e0, w0t) + _dot_t(e1, w0b)
    h_o = _dot_t(o0, w0t) + _dot_t(o1, w0b)
    n = float(NPER)
    m = (jnp.sum(h_e, 0, keepdims=True) + jnp.sum(h_o, 0, keepdims=True)) / n
    hm_e = h_e - m
    hm_o = h_o - m
    v = (jnp.sum(hm_e * hm_e, 0, keepdims=True)
         + jnp.sum(hm_o * hm_o, 0, keepdims=True)) / n
    r0 = lax.rsqrt(v + 1e-5) * g0_ref[...]
    x1_e = jnp.maximum(hm_e * r0 + b0_ref[...], 0.0)
    x1_o = jnp.maximum(hm_o * r0 + b0_ref[...], 0.0)
    w1 = w1_ref[...]
    h2_e = _dot_t(x1_e, w1)
    h2_o = _dot_t(x1_o, w1)
    m2 = (jnp.sum(h2_e, 0, keepdims=True)
          + jnp.sum(h2_o, 0, keepdims=True)) / n
    hm2_e = h2_e - m2
    hm2_o = h2_o - m2
    v2 = (jnp.sum(hm2_e * hm2_e, 0, keepdims=True)
          + jnp.sum(hm2_o * hm2_o, 0, keepdims=True)) / n
    r1 = lax.rsqrt(v2 + 1e-5) * g1_ref[...]
    y_e = jnp.maximum(hm2_e * r1 + b1_ref[...], 0.0)
    y_o = jnp.maximum(hm2_o * r1 + b1_ref[...], 0.0)
    sc = 1.0 + eps_ref[0, 0]
    xr = xres_ref[...]
    out_e = sc * xr[:, 0:NHID] + y_e
    out_o = sc * xr[:, NHID:2 * NHID] + y_o
    full_ref[...] = jnp.concatenate([out_e, out_o], axis=1)
    split_ref[0:SHLF, :] = jnp.concatenate(
        [out_e[:, 0:HALF], out_o[:, 0:HALF]], axis=1)
    split_ref[SHLF:NPER, :] = jnp.concatenate(
        [out_e[:, HALF:NHID], out_o[:, HALF:NHID]], axis=1)


_mlp = pl.pallas_call(
    _mlp_body,
    out_shape=(jax.ShapeDtypeStruct((NPER, NHID), jnp.float32),
               jax.ShapeDtypeStruct((SHLF, 2 * NHID), jnp.float32)),
    in_specs=[pl.BlockSpec(memory_space=pltpu.VMEM)] * 8
             + [pl.BlockSpec(memory_space=pltpu.SMEM)],
)


def _split(x):
    n = x.shape[0]
    return x.reshape(n, 2, HALF).transpose(1, 0, 2).reshape(2 * n, HALF)


def _phase(tab_b, x_res_full, src_idx, dst_idx, W0, g0, b0, W1, g1, b1,
           eps):
    tab3 = tab_b.reshape(NC, NPER, HALF)
    dst2 = dst_idx.reshape(NS * NCHUNK, CH)
    agg = _sc_segsum(tab3, src_idx, dst2)
    split_b, full_v = _mlp(agg.reshape(NPER, NHID),
                           x_res_full.reshape(SHLF, 2 * NHID),
                           W0, g0.reshape(1, NHID), b0.reshape(1, NHID),
                           W1, g1.reshape(1, NHID), b1.reshape(1, NHID),
                           eps.reshape(1, 1))
    return split_b, full_v.reshape(NPER, NHID)


def kernel(xs, k_batch, bipartites, bw_W0, bw_g0, bw_b0, bw_W1, bw_g1, bw_b1,
           fw_W0, fw_g0, fw_b0, fw_W1, fw_g1, fw_b1, bw_eps, fw_eps):
    L0 = xs[0:NPER]
    L1 = xs[NPER:2 * NPER]
    L2 = xs[2 * NPER:3 * NPER]
    ei0 = bipartites[0]
    ei1 = bipartites[1]

    nl1_s, nl1_f = _phase(_split(L2), L1, ei1[1], ei1[0],
                          bw_W0[1], bw_g0[1], bw_b0[1], bw_W1[1], bw_g1[1],
                          bw_b1[1], bw_eps[1])
    nl0_s, nl0_f = _phase(nl1_s, L0, ei0[1], ei0[0],
                          bw_W0[0], bw_g0[0], bw_b0[0], bw_W1[0], bw_g1[0],
                          bw_b1[0], bw_eps[0])
    nl1b_s, nl1b_f = _phase(nl0_s, nl1_f, ei0[0], ei0[1],
                            fw_W0[0], fw_g0[0], fw_b0[0], fw_W1[0], fw_g1[0],
                            fw_b1[0], fw_eps[0])
    _, nl2_f = _phase(nl1b_s, L2, ei1[0], ei1[1],
                      fw_W0[1], fw_g0[1], fw_b0[1], fw_W1[1], fw_g1[1],
                      fw_b1[1], fw_eps[1])
    return jnp.concatenate([nl0_f, nl1b_f, nl2_f], axis=0)


# speedup vs baseline: 1.2727x; 1.0297x over previous
import functools

import jax
import jax.numpy as jnp
from jax import lax
from jax.experimental import pallas as pl
from jax.experimental.pallas import tpu as pltpu
from jax.experimental.pallas import tpu_sc as plsc

NHID = 128
HALF = 64
NPER = 16384
E = 262144
NC = 2
NS = 16
CH = 128
EDGES_PER_TILE = E // NS
NCHUNK = EDGES_PER_TILE // CH
ROWS_PER_TILE = NPER // NS


NBUF = 4


def _sc_segsum_body(table, src_hbm, dst_hbm, out, sidx, didx, rows, acc,
                    gsem, ssem):
    c = lax.axis_index("c")
    s = lax.axis_index("s")

    pltpu.sync_copy(src_hbm.at[pl.ds(s * EDGES_PER_TILE, EDGES_PER_TILE)],
                    sidx)
    pltpu.sync_copy(dst_hbm.at[pl.ds(s * NCHUNK, NCHUNK)], didx)
    tabc = table.at[c]
    for b in range(NBUF - 1):
        pltpu.async_copy(tabc.at[sidx.at[pl.ds(b * CH, CH)]], rows[b],
                         gsem[b])

    zero = jnp.zeros((16,), jnp.float32)
    zbuf = rows[NBUF - 1]

    def zrow(i, carry):
        for j in range(HALF // 16):
            zbuf[i, pl.ds(j * 16, 16)] = zero
        return carry

    lax.fori_loop(0, CH, zrow, 0)
    for k in range(ROWS_PER_TILE // CH):
        pltpu.sync_copy(zbuf, acc.at[pl.ds(s * ROWS_PER_TILE + k * CH, CH)])
    plsc.subcore_barrier()

    def step(k2, carry):
        for b in range(NBUF):
            k = k2 * NBUF + b
            pltpu.make_async_copy(tabc.at[sidx.at[pl.ds(0, CH)]], rows[b],
                                  gsem[b]).wait()
            pltpu.async_copy(rows[b], acc.at[didx.at[k]], ssem[b], add=True)
            b3 = (b - 1) % NBUF
            kn = k + NBUF - 1

            @pl.when(kn < NCHUNK)
            def _():
                @pl.when(k >= 1)
                def _():
                    pltpu.make_async_copy(rows[b3], acc.at[didx.at[0]],
                                          ssem[b3]).wait()
                pltpu.async_copy(tabc.at[sidx.at[pl.ds(kn * CH, CH)]],
                                 rows[b3], gsem[b3])

        return carry

    lax.fori_loop(0, NCHUNK // NBUF, step, 0)
    for b in range(NBUF):
        pltpu.make_async_copy(rows[b], acc.at[didx.at[0]], ssem[b]).wait()
    plsc.subcore_barrier()
    pltpu.sync_copy(acc.at[pl.ds(s * ROWS_PER_TILE, ROWS_PER_TILE)],
                    out.at[pl.ds(c * NPER + s * ROWS_PER_TILE,
                                 ROWS_PER_TILE)])


@functools.cache
def _get_sc_segsum():
    return functools.partial(
        pl.kernel,
        mesh=plsc.VectorSubcoreMesh(core_axis_name="c", subcore_axis_name="s"),
        out_type=jax.ShapeDtypeStruct((NC * NPER, HALF), jnp.float32),
        scratch_types=[
            pltpu.VMEM((EDGES_PER_TILE,), jnp.int32),
            pltpu.VMEM((NCHUNK, CH), jnp.int32),
            [pltpu.VMEM((CH, HALF), jnp.float32)] * NBUF,
            pltpu.VMEM_SHARED((NPER, HALF), jnp.float32),
            [pltpu.SemaphoreType.DMA] * NBUF,
            [pltpu.SemaphoreType.DMA] * NBUF,
        ],
        compiler_params=pltpu.CompilerParams(use_tc_tiling_on_sc=False),
    )(_sc_segsum_body)


def _sc_segsum(table, src_idx2, dst_idx2):
    return _get_sc_segsum()(table, src_idx2, dst_idx2)


SHLF = NPER // 2


def _dot_t(x, w):
    return lax.dot_general(x, w, (((1,), (1,)), ((), ())),
                           preferred_element_type=jnp.float32)


def _mlp_core(a_ref, xres_ref, w0_ref, g0_ref, b0_ref, w1_ref, g1_ref,
              b1_ref, eps_ref):
    e0 = a_ref[0:SHLF, 0:HALF]
    o0 = a_ref[0:SHLF, HALF:NHID]
    e1 = a_ref[SHLF:NPER, 0:HALF]
    o1 = a_ref[SHLF:NPER, HALF:NHID]
    w0 = w0_ref[...]
    w0t = w0[:, 0:HALF]
    w0b = w0[:, HALF:NHID]
    h_e = _dot_t(e0, w0t) + _dot_t(e1, w0b)
    h_o = _dot_t(o0, w0t) + _dot_t(o1, w0b)
    n = float(NPER)
    m = (jnp.sum(h_e, 0, keepdims=True) + jnp.sum(h_o, 0, keepdims=True)) / n
    hm_e = h_e - m
    hm_o = h_o - m
    v = (jnp.sum(hm_e * hm_e, 0, keepdims=True)
         + jnp.sum(hm_o * hm_o, 0, keepdims=True)) / n
    r0 = lax.rsqrt(v + 1e-5) * g0_ref[...]
    x1_e = jnp.maximum(hm_e * r0 + b0_ref[...], 0.0)
    x1_o = jnp.maximum(hm_o * r0 + b0_ref[...], 0.0)
    w1 = w1_ref[...]
    h2_e = _dot_t(x1_e, w1)
    h2_o = _dot_t(x1_o, w1)
    m2 = (jnp.sum(h2_e, 0, keepdims=True)
          + jnp.sum(h2_o, 0, keepdims=True)) / n
    hm2_e = h2_e - m2
    hm2_o = h2_o - m2
    v2 = (jnp.sum(hm2_e * hm2_e, 0, keepdims=True)
          + jnp.sum(hm2_o * hm2_o, 0, keepdims=True)) / n
    r1 = lax.rsqrt(v2 + 1e-5) * g1_ref[...]
    y_e = jnp.maximum(hm2_e * r1 + b1_ref[...], 0.0)
    y_o = jnp.maximum(hm2_o * r1 + b1_ref[...], 0.0)
    sc = 1.0 + eps_ref[0, 0]
    xr = xres_ref[...]
    out_e = sc * xr[:, 0:NHID] + y_e
    out_o = sc * xr[:, NHID:2 * NHID] + y_o
    return out_e, out_o


def _write_split(split_ref, out_e, out_o):
    split_ref[0:SHLF, :] = jnp.concatenate(
        [out_e[:, 0:HALF], out_o[:, 0:HALF]], axis=1)
    split_ref[SHLF:NPER, :] = jnp.concatenate(
        [out_e[:, HALF:NHID], out_o[:, HALF:NHID]], axis=1)


def _mlp_body(a_ref, xres_ref, w0_ref, g0_ref, b0_ref, w1_ref, g1_ref,
              b1_ref, eps_ref, split_ref, full_ref):
    out_e, out_o = _mlp_core(a_ref, xres_ref, w0_ref, g0_ref, b0_ref,
                             w1_ref, g1_ref, b1_ref, eps_ref)
    full_ref[...] = jnp.concatenate([out_e, out_o], axis=1)
    _write_split(split_ref, out_e, out_o)


def _mlp_mid_body(lvl, a_ref, xres_ref, w0_ref, g0_ref, b0_ref, w1_ref,
                  g1_ref, b1_ref, eps_ref, split_ref, xs_ref, fbuf, sem):
    out_e, out_o = _mlp_core(a_ref, xres_ref, w0_ref, g0_ref, b0_ref,
                             w1_ref, g1_ref, b1_ref, eps_ref)
    _write_split(split_ref, out_e, out_o)
    fbuf[...] = jnp.concatenate([out_e, out_o], axis=1)
    pltpu.async_copy(fbuf, xs_ref.at[pl.ds(lvl * SHLF, SHLF)], sem).wait()


def _mlp_last_body(lvl, a_ref, xres_ref, w0_ref, g0_ref, b0_ref, w1_ref,
                   g1_ref, b1_ref, eps_ref, xsin_ref, xs_ref, fbuf, sem):
    out_e, out_o = _mlp_core(a_ref, xres_ref, w0_ref, g0_ref, b0_ref,
                             w1_ref, g1_ref, b1_ref, eps_ref)
    fbuf[...] = jnp.concatenate([out_e, out_o], axis=1)
    pltpu.async_copy(fbuf, xs_ref.at[pl.ds(lvl * SHLF, SHLF)], sem).wait()


def _mlp_mid2_body(lvl, a_ref, xres_ref, w0_ref, g0_ref, b0_ref, w1_ref,
                   g1_ref, b1_ref, eps_ref, xsin_ref, split_ref, xs_ref,
                   fbuf, sem):
    _mlp_mid_body(lvl, a_ref, xres_ref, w0_ref, g0_ref, b0_ref, w1_ref,
                  g1_ref, b1_ref, eps_ref, split_ref, xs_ref, fbuf, sem)


_SPECS9 = ([pl.BlockSpec(memory_space=pltpu.VMEM)] * 8
           + [pl.BlockSpec(memory_space=pltpu.SMEM)])
_FSCRATCH = [pltpu.VMEM((SHLF, 2 * NHID), jnp.float32),
             pltpu.SemaphoreType.DMA]

_mlp = pl.pallas_call(
    _mlp_body,
    out_shape=(jax.ShapeDtypeStruct((NPER, NHID), jnp.float32),
               jax.ShapeDtypeStruct((SHLF, 2 * NHID), jnp.float32)),
    in_specs=_SPECS9,
)

_mlp_first_final = pl.pallas_call(
    functools.partial(_mlp_mid_body, 0),
    out_shape=(jax.ShapeDtypeStruct((NPER, NHID), jnp.float32),
               jax.ShapeDtypeStruct((3 * SHLF, 2 * NHID), jnp.float32)),
    in_specs=_SPECS9,
    out_specs=(pl.BlockSpec(memory_space=pltpu.VMEM),
               pl.BlockSpec(memory_space=pl.ANY)),
    scratch_shapes=_FSCRATCH,
)

_mlp_mid_final = pl.pallas_call(
    functools.partial(_mlp_mid2_body, 1),
    out_shape=(jax.ShapeDtypeStruct((NPER, NHID), jnp.float32),
               jax.ShapeDtypeStruct((3 * SHLF, 2 * NHID), jnp.float32)),
    in_specs=_SPECS9 + [pl.BlockSpec(memory_space=pl.ANY)],
    out_specs=(pl.BlockSpec(memory_space=pltpu.VMEM),
               pl.BlockSpec(memory_space=pl.ANY)),
    input_output_aliases={9: 1},
    scratch_shapes=_FSCRATCH,
)

_mlp_last_final = pl.pallas_call(
    functools.partial(_mlp_last_body, 2),
    out_shape=jax.ShapeDtypeStruct((3 * SHLF, 2 * NHID), jnp.float32),
    in_specs=_SPECS9 + [pl.BlockSpec(memory_space=pl.ANY)],
    out_specs=pl.BlockSpec(memory_space=pl.ANY),
    input_output_aliases={9: 0},
    scratch_shapes=_FSCRATCH,
)


def _split_body(v_ref, out_ref):
    v = v_ref[...]
    out_ref[0:SHLF, :] = jnp.concatenate(
        [v[:, 0:HALF], v[:, NHID:NHID + HALF]], axis=1)
    out_ref[SHLF:NPER, :] = jnp.concatenate(
        [v[:, HALF:NHID], v[:, NHID + HALF:2 * NHID]], axis=1)


_split_k = pl.pallas_call(
    _split_body,
    out_shape=jax.ShapeDtypeStruct((NPER, NHID), jnp.float32),
)


def _seg(tab_b, src_idx, dst_idx):
    agg = _sc_segsum(tab_b.reshape(NC, NPER, HALF), src_idx,
                     dst_idx.reshape(NS * NCHUNK, CH))
    return agg.reshape(NPER, NHID)


def _wargs(W0, g0, b0, W1, g1, b1, eps):
    return (W0, g0.reshape(1, NHID), b0.reshape(1, NHID),
            W1, g1.reshape(1, NHID), b1.reshape(1, NHID), eps.reshape(1, 1))


def kernel(xs, k_batch, bipartites, bw_W0, bw_g0, bw_b0, bw_W1, bw_g1, bw_b1,
           fw_W0, fw_g0, fw_b0, fw_W1, fw_g1, fw_b1, bw_eps, fw_eps):
    pv = lambda x: x.reshape(SHLF, 2 * NHID)
    L0 = pv(xs[0:NPER])
    L1 = pv(xs[NPER:2 * NPER])
    L2 = pv(xs[2 * NPER:3 * NPER])
    ei0 = bipartites[0]
    ei1 = bipartites[1]

    agg = _seg(_split_k(L2), ei1[1], ei1[0])
    nl1_s, nl1_f = _mlp(agg, L1, *_wargs(bw_W0[1], bw_g0[1], bw_b0[1],
                                         bw_W1[1], bw_g1[1], bw_b1[1],
                                         bw_eps[1]))
    agg = _seg(nl1_s, ei0[1], ei0[0])
    nl0_s, xsb = _mlp_first_final(agg, L0, *_wargs(bw_W0[0], bw_g0[0],
                                                   bw_b0[0], bw_W1[0],
                                                   bw_g1[0], bw_b1[0],
                                                   bw_eps[0]))
    agg = _seg(nl0_s, ei0[0], ei0[1])
    nl1b_s, xsb = _mlp_mid_final(agg, nl1_f, *_wargs(fw_W0[0], fw_g0[0],
                                                     fw_b0[0], fw_W1[0],
                                                     fw_g1[0], fw_b1[0],
                                                     fw_eps[0]), xsb)
    agg = _seg(nl1b_s, ei1[0], ei1[1])
    xsb = _mlp_last_final(agg, L2, *_wargs(fw_W0[1], fw_g0[1], fw_b0[1],
                                           fw_W1[1], fw_g1[1], fw_b1[1],
                                           fw_eps[1]), xsb)
    return xsb.reshape(3 * NPER, NHID)
